# Initial kernel scaffold; baseline (speedup 1.0000x reference)
import jax
import jax.numpy as jnp
from jax.experimental import pallas as pl


def kernel(x, pos_emb):
    raise NotImplementedError("write your pallas kernel here")


# speedup vs baseline: 3.7251x; 3.7251x over previous
import jax
import jax.numpy as jnp
from jax.experimental import pallas as pl


def kernel(x, pos_emb):
    batch, seq, d = x.shape
    R = 512
    nb = seq // R

    def body(pe_ref, o_ref):
        o_ref[0] = pe_ref[...]

    return pl.pallas_call(
        body,
        grid=(nb, batch),
        in_specs=[pl.BlockSpec((R, d), lambda i, j: (i, 0))],
        out_specs=pl.BlockSpec((1, R, d), lambda i, j: (j, i, 0)),
        out_shape=jax.ShapeDtypeStruct((batch, seq, d), pos_emb.dtype),
    )(pos_emb)


# speedup vs baseline: 4.5896x; 1.2321x over previous
import functools

import jax
import jax.numpy as jnp
from jax import lax
from jax.experimental import pallas as pl
from jax.experimental.pallas import tpu as pltpu
from jax.experimental.pallas import tpu_sc as plsc


def kernel(x, pos_emb):
    batch, seq, d = x.shape
    V, D = pos_emb.shape
    info = plsc.get_sparse_core_info()
    nc, ns = info.num_cores, info.num_subcores
    nw = nc * ns
    rows_per_w = V // nw
    mesh = plsc.VectorSubcoreMesh(core_axis_name="c", subcore_axis_name="s")

    @functools.partial(
        pl.kernel,
        mesh=mesh,
        out_type=jax.ShapeDtypeStruct((batch * V, D), pos_emb.dtype),
        scratch_types=[
            pltpu.VMEM((rows_per_w, D), jnp.float32),
            pltpu.SemaphoreType.DMA,
        ],
    )
    def bcast(pe_hbm, out_hbm, buf, sem):
        wid = lax.axis_index("s") * nc + lax.axis_index("c")
        base = wid * rows_per_w
        pltpu.sync_copy(pe_hbm.at[pl.ds(base, rows_per_w)], buf)
        copies = [
            pltpu.async_copy(buf, out_hbm.at[pl.ds(b * V + base, rows_per_w)], sem)
            for b in range(batch)
        ]
        for c in copies:
            c.wait()

    return bcast(pos_emb).reshape(batch, V, D)


# speedup vs baseline: 4.6074x; 1.0039x over previous
import functools

import jax
import jax.numpy as jnp
from jax import lax
from jax.experimental import pallas as pl
from jax.experimental.pallas import tpu as pltpu
from jax.experimental.pallas import tpu_sc as plsc


def kernel(x, pos_emb):
    batch, seq, d = x.shape
    V, D = pos_emb.shape
    info = plsc.get_sparse_core_info()
    nc, ns = info.num_cores, info.num_subcores
    nw = nc * ns
    rows_per_w = V // nw
    n_chunks = 4
    crows = rows_per_w // n_chunks
    mesh = plsc.VectorSubcoreMesh(core_axis_name="c", subcore_axis_name="s")

    @functools.partial(
        pl.kernel,
        mesh=mesh,
        out_type=jax.ShapeDtypeStruct((batch * V, D), pos_emb.dtype),
        scratch_types=[
            pltpu.VMEM((rows_per_w, D), jnp.float32),
            pltpu.SemaphoreType.DMA,
            pltpu.SemaphoreType.DMA,
        ],
    )
    def bcast(pe_hbm, out_hbm, buf, rsem, wsem):
        wid = lax.axis_index("s") * nc + lax.axis_index("c")
        base = wid * rows_per_w
        reads = [
            pltpu.async_copy(
                pe_hbm.at[pl.ds(base + c * crows, crows)],
                buf.at[pl.ds(c * crows, crows)],
                rsem,
            )
            for c in range(n_chunks)
        ]
        writes = []
        for c in range(n_chunks):
            reads[c].wait()
            writes += [
                pltpu.async_copy(
                    buf.at[pl.ds(c * crows, crows)],
                    out_hbm.at[pl.ds(b * V + base + c * crows, crows)],
                    wsem,
                )
                for b in range(batch)
            ]
        for w in writes:
            w.wait()

    return bcast(pos_emb).reshape(batch, V, D)


# speedup vs baseline: 10.1770x; 2.2089x over previous
import jax
import jax.numpy as jnp
from jax.experimental import pallas as pl
from jax.experimental.pallas import tpu as pltpu


def kernel(x, pos_emb):
    batch, seq, d = x.shape
    V, D = pos_emb.shape

    def body(pe_hbm, o_hbm, vbuf, rsem, wsem):
        rd = pltpu.make_async_copy(pe_hbm, vbuf, rsem)
        rd.start()
        rd.wait()
        ws = []
        for b in range(batch):
            w = pltpu.make_async_copy(vbuf, o_hbm.at[b], wsem)
            w.start()
            ws.append(w)
        for w in ws:
            w.wait()

    return pl.pallas_call(
        body,
        in_specs=[pl.BlockSpec(memory_space=pl.ANY)],
        out_specs=pl.BlockSpec(memory_space=pl.ANY),
        out_shape=jax.ShapeDtypeStruct((batch, V, D), pos_emb.dtype),
        scratch_shapes=[
            pltpu.VMEM((V, D), jnp.float32),
            pltpu.SemaphoreType.DMA,
            pltpu.SemaphoreType.DMA,
        ],
    )(pos_emb)
